# baseline (speedup 1.0000x reference)
import functools
import math

import jax
import jax.numpy as jnp
from jax import lax
from jax.experimental import pallas as pl
from jax.experimental.pallas import tpu as pltpu
from jax.experimental.pallas import tpu_sc as plsc

D_EMB = 64
LANES = 16
BLK = 128
SCALE = math.sqrt(D_EMB)


@jax.jit
def _emb_lookup(xt, table):
    n_j, n_i = xt.shape
    info = plsc.get_sparse_core_info()
    nc, ns = info.num_cores, info.num_subcores
    nw = nc * ns
    iw = n_i // nw
    n_blk = iw // BLK

    mesh = plsc.VectorSubcoreMesh(core_axis_name="c", subcore_axis_name="s")

    @functools.partial(
        pl.kernel,
        mesh=mesh,
        out_type=jax.ShapeDtypeStruct((n_j, D_EMB, n_i), jnp.float32),
        scratch_types=[
            pltpu.VMEM((iw,), jnp.int32),
            pltpu.VMEM((iw, D_EMB), jnp.float32),
            pltpu.VMEM((D_EMB, iw), jnp.float32),
            pltpu.SemaphoreType.DMA,
        ],
        compiler_params=pltpu.CompilerParams(
            use_tc_tiling_on_sc=False, needs_layout_passes=False
        ),
    )
    def k(xt_hbm, table_hbm, out_hbm, idx_v, rows_v, trans_v, sem):
        wid = lax.axis_index("s") * nc + lax.axis_index("c")
        i0 = wid * iw
        iota = lax.iota(jnp.int32, LANES)

        def j_body(j, carry):
            pltpu.sync_copy(xt_hbm.at[j, pl.ds(i0, iw)], idx_v)
            handles = [
                pltpu.async_copy(
                    table_hbm.at[idx_v.at[pl.ds(b * BLK, BLK)]],
                    rows_v.at[pl.ds(b * BLK, BLK)],
                    sem,
                )
                for b in range(n_blk)
            ]
            for h in handles:
                h.wait()

            def d_body(d, c2):
                col = jnp.full((LANES,), 0, jnp.int32) + d
                for g in range(iw // LANES):
                    rows16 = iota + (g * LANES)
                    v = plsc.load_gather(rows_v, [rows16, col])
                    trans_v[d, pl.ds(g * LANES, LANES)] = v * SCALE
                return c2

            lax.fori_loop(0, D_EMB, d_body, 0)
            pltpu.sync_copy(trans_v, out_hbm.at[j, :, pl.ds(i0, iw)])
            return carry

        lax.fori_loop(0, n_j, j_body, 0)

    return k(xt, table)


def kernel(x, table):
    out = _emb_lookup(jnp.transpose(x), table)
    return jnp.transpose(out, (2, 0, 1))


# speedup vs baseline: 1.1343x; 1.1343x over previous
import functools
import math

import jax
import jax.numpy as jnp
from jax import lax
from jax.experimental import pallas as pl
from jax.experimental.pallas import tpu as pltpu
from jax.experimental.pallas import tpu_sc as plsc

D_EMB = 64
LANES = 16
BLK = 128
SCALE = math.sqrt(D_EMB)


@jax.jit
def _emb_lookup(xt, table):
    n_j, n_i = xt.shape
    info = plsc.get_sparse_core_info()
    nc, ns = info.num_cores, info.num_subcores
    nw = nc * ns
    iw = n_i // nw
    n_blk = iw // BLK

    mesh = plsc.VectorSubcoreMesh(core_axis_name="c", subcore_axis_name="s")

    @functools.partial(
        pl.kernel,
        mesh=mesh,
        out_type=jax.ShapeDtypeStruct((n_j, D_EMB, n_i), jnp.float32),
        scratch_types=[
            pltpu.VMEM((iw,), jnp.int32),
            pltpu.VMEM((iw, D_EMB), jnp.float32),
            pltpu.VMEM((D_EMB, iw), jnp.float32),
            pltpu.SemaphoreType.DMA,
        ],
        compiler_params=pltpu.CompilerParams(
            use_tc_tiling_on_sc=False, needs_layout_passes=False
        ),
    )
    def k(xt_hbm, table_hbm, out_hbm, idx_v, rows_v, trans_v, sem):
        wid = lax.axis_index("s") * nc + lax.axis_index("c")
        i0 = wid * iw
        iota = lax.iota(jnp.int32, LANES)

        def j_body(j, carry):
            pltpu.sync_copy(xt_hbm.at[j, pl.ds(i0, iw)], idx_v)
            handles = [
                pltpu.async_copy(
                    table_hbm.at[idx_v.at[pl.ds(b * BLK, BLK)]],
                    rows_v.at[pl.ds(b * BLK, BLK)],
                    sem,
                )
                for b in range(n_blk)
            ]
            for h in handles:
                h.wait()

            def r_body(r4, c2):
                r = r4 * 4
                for rr in range(4):
                    col = jnp.full((LANES,), 0, jnp.int32) + (r + rr)
                    for c in range(D_EMB // LANES):
                        v = rows_v[r + rr, pl.ds(c * LANES, LANES)]
                        plsc.store_scatter(
                            trans_v, [iota + (c * LANES), col], v * SCALE
                        )
                return c2

            lax.fori_loop(0, iw // 4, r_body, 0)
            pltpu.sync_copy(trans_v, out_hbm.at[j, :, pl.ds(i0, iw)])
            return carry

        lax.fori_loop(0, n_j, j_body, 0)

    return k(xt, table)


def kernel(x, table):
    out = _emb_lookup(jnp.transpose(x), table)
    return jnp.transpose(out, (2, 0, 1))


# speedup vs baseline: 2.2102x; 1.9485x over previous
import functools
import math

import jax
import jax.numpy as jnp
from jax import lax
from jax.experimental import pallas as pl
from jax.experimental.pallas import tpu as pltpu
from jax.experimental.pallas import tpu_sc as plsc

D_EMB = 64
LANES = 16
BLK = 128
PAD = 15
SCALE = math.sqrt(D_EMB)


@jax.jit
def _emb_lookup(xt, table):
    n_j, n_i = xt.shape
    info = plsc.get_sparse_core_info()
    nc, ns = info.num_cores, info.num_subcores
    nw = nc * ns
    iw = n_i // nw
    n_blk = iw // BLK
    n_ch = D_EMB // LANES

    mesh = plsc.VectorSubcoreMesh(core_axis_name="c", subcore_axis_name="s")

    @functools.partial(
        pl.kernel,
        mesh=mesh,
        out_type=jax.ShapeDtypeStruct((n_j, D_EMB, n_i), jnp.float32),
        scratch_types=[
            pltpu.VMEM((2, iw), jnp.int32),
            pltpu.VMEM((2, iw, D_EMB), jnp.float32),
            pltpu.VMEM((D_EMB, iw + PAD), jnp.float32),
            pltpu.SemaphoreType.DMA,
            pltpu.SemaphoreType.DMA,
        ],
        compiler_params=pltpu.CompilerParams(
            use_tc_tiling_on_sc=False, needs_layout_passes=False
        ),
    )
    def k(xt_hbm, table_hbm, out_hbm, idx_v, rows_v, trans_v, sem0, sem1):
        wid = lax.axis_index("s") * nc + lax.axis_index("c")
        i0 = wid * iw
        iota = lax.iota(jnp.int32, LANES)
        sems = (sem0, sem1)

        def fire(j, buf, sem):
            pltpu.sync_copy(xt_hbm.at[j, pl.ds(i0, iw)], idx_v.at[buf])
            for b in range(n_blk):
                pltpu.async_copy(
                    table_hbm.at[idx_v.at[buf, pl.ds(b * BLK, BLK)]],
                    rows_v.at[buf, pl.ds(b * BLK, BLK)],
                    sem,
                )

        def drain(buf, sem):
            for b in range(n_blk):
                pltpu.make_async_copy(
                    table_hbm.at[idx_v.at[buf, pl.ds(b * BLK, BLK)]],
                    rows_v.at[buf, pl.ds(b * BLK, BLK)],
                    sem,
                ).wait()

        def process(j, buf):
            @plsc.parallel_loop(0, iw, step=4, unroll=2)
            def transpose_body(r):
                vals = [
                    rows_v[buf, r + rr, pl.ds(c * LANES, LANES)] * SCALE
                    for rr in range(4)
                    for c in range(n_ch)
                ]
                for rr in range(4):
                    col = jnp.full((LANES,), 0, jnp.int32) + (r + rr)
                    for c in range(n_ch):
                        plsc.store_scatter(
                            trans_v,
                            [iota + (c * LANES), col],
                            vals[rr * n_ch + c],
                        )

            pltpu.sync_copy(
                trans_v.at[:, pl.ds(0, iw)], out_hbm.at[j, :, pl.ds(i0, iw)]
            )

        fire(0, 0, sems[0])

        def pair_body(p, carry):
            for par in range(2):
                j = p * 2 + par
                nxt = j + 1

                @pl.when(nxt < n_j)
                def _():
                    fire(nxt, 1 - par, sems[1 - par])

                drain(par, sems[par])
                process(j, par)
            return carry

        lax.fori_loop(0, n_j // 2, pair_body, 0)

    return k(xt, table)


def kernel(x, table):
    out = _emb_lookup(jnp.transpose(x), table)
    return jnp.transpose(out, (2, 0, 1))
